# baseline (speedup 1.0000x reference)
import functools
import math

import jax
import jax.numpy as jnp
from jax import lax
from jax.experimental import pallas as pl
from jax.experimental.pallas import tpu as pltpu
from jax.experimental.pallas import tpu_sc as plsc

N_NODES = 10000
E = 640000
K = 64
NC = 2
NS = 16
NW = NC * NS

_BE = 25600
_NB1 = 4
_E1 = _NB1 * _BE
_E2 = E - _E1

_mesh = plsc.VectorSubcoreMesh(core_axis_name="c", subcore_axis_name="s")


def _make_sqdist_sc(e0, n_edges):
    epw = n_edges // NW
    tiles = (epw + 96 + 127) // 128

    @functools.partial(
        pl.kernel,
        mesh=_mesh,
        compiler_params=pltpu.CompilerParams(needs_layout_passes=False),
        out_type=jax.ShapeDtypeStruct((n_edges,), jnp.float32),
        scratch_types=[
            pltpu.VMEM((N_NODES,), jnp.float32),
            pltpu.VMEM((N_NODES,), jnp.float32),
            pltpu.VMEM((N_NODES,), jnp.float32),
            pltpu.VMEM((2 * 128 * tiles,), jnp.int32),
            pltpu.VMEM((epw,), jnp.float32),
            pltpu.SemaphoreType.DMA,
        ],
    )
    def _sqdist(posT_hbm, eit_hbm, out_hbm, px_v, py_v, pz_v, ei_v, out_v, sem):
        wid = lax.axis_index("s") * NC + lax.axis_index("c")
        base = wid * epw
        gbase = e0 + base
        t0 = gbase // 128
        ls = gbase - 128 * t0
        cps = [
            pltpu.async_copy(posT_hbm.at[pl.ds(0, N_NODES)], px_v, sem),
            pltpu.async_copy(posT_hbm.at[pl.ds(N_NODES, N_NODES)], py_v, sem),
            pltpu.async_copy(posT_hbm.at[pl.ds(2 * N_NODES, N_NODES)], pz_v, sem),
            pltpu.async_copy(eit_hbm.at[pl.ds(256 * t0, 256 * tiles)], ei_v, sem),
        ]
        for cp in cps:
            cp.wait()

        @plsc.parallel_loop(0, epw, step=16, unroll=8)
        def _body(l):
            le = ls + l
            b = le >> 7
            off = (b << 8) + (le - (b << 7))
            si = ei_v[pl.ds(off, 16)]
            di = ei_v[pl.ds(off + 128, 16)]
            dx = plsc.load_gather(px_v, [si]) - plsc.load_gather(px_v, [di])
            dy = plsc.load_gather(py_v, [si]) - plsc.load_gather(py_v, [di])
            dz = plsc.load_gather(pz_v, [si]) - plsc.load_gather(pz_v, [di])
            out_v[pl.ds(l, 16)] = dx * dx + dy * dy + dz * dz
        pltpu.sync_copy(out_v, out_hbm.at[pl.ds(base, epw)])

    return _sqdist


_sqdist_sc1 = _make_sqdist_sc(0, _E1)
_sqdist_sc2 = _make_sqdist_sc(_E1, _E2)

_R = _BE // 128


def _rbf_body(s_ref, mu_ref, ls_ref, out_ref):
    ls = ls_ref[...]
    sig = jnp.logaddexp(ls, 0.0)
    a = (-0.5 * math.log2(math.e)) / sig
    c = -1.0 / jnp.sqrt(2.0 * math.pi * sig)
    mub = jnp.broadcast_to(mu_ref[...], (K, 128))
    ab = jnp.broadcast_to(a, (K, 128))
    cb = jnp.broadcast_to(c, (K, 128))
    for r in range(_R):
        srow = s_ref[r:r + 1, :]
        d = srow * lax.rsqrt(srow + 1e-37)
        db = jnp.broadcast_to(d, (K, 128))
        diff = db - mub
        out_ref[:, r * 128:(r + 1) * 128] = cb * jnp.exp2(ab * (diff * diff))


def _rbf_phase1(s_ref, mu_ref, ls_ref, out_ref):
    _rbf_body(s_ref, mu_ref, ls_ref, out_ref)


def _rbf_phase2(s_ref, mu_ref, ls_ref, prev_ref, out_ref):
    del prev_ref
    _rbf_body(s_ref, mu_ref, ls_ref, out_ref)


_rbf_call1 = pl.pallas_call(
    _rbf_phase1,
    grid=(_NB1,),
    in_specs=[
        pl.BlockSpec((_R, 128), lambda i: (i, 0)),
        pl.BlockSpec((K, 1), lambda i: (0, 0)),
        pl.BlockSpec((K, 1), lambda i: (0, 0)),
    ],
    out_specs=pl.BlockSpec((K, _BE), lambda i: (0, i)),
    out_shape=jax.ShapeDtypeStruct((K, E), jnp.float32),
)

_rbf_call2 = pl.pallas_call(
    _rbf_phase2,
    grid=(_E2 // _BE,),
    in_specs=[
        pl.BlockSpec((_R, 128), lambda i: (i, 0)),
        pl.BlockSpec((K, 1), lambda i: (0, 0)),
        pl.BlockSpec((K, 1), lambda i: (0, 0)),
        pl.BlockSpec((8, 128), lambda i: (0, 0)),
    ],
    out_specs=pl.BlockSpec((K, _BE), lambda i: (0, i + _NB1)),
    out_shape=jax.ShapeDtypeStruct((K, E), jnp.float32),
    input_output_aliases={3: 0},
)


def kernel(edge_index, pos_matrix, mu, log_sigma):
    ei = edge_index.astype(jnp.int32)
    eit = ei.reshape(2, E // 128, 128).swapaxes(0, 1).reshape(2 * E)
    posT = pos_matrix.T.reshape(3 * N_NODES)
    mu1 = mu.reshape(K, 1)
    ls1 = log_sigma.reshape(K, 1)
    s1 = _sqdist_sc1(posT, eit)
    s2 = _sqdist_sc2(posT, eit)
    out1 = _rbf_call1(s1.reshape(_E1 // 128, 128), mu1, ls1)
    out2 = _rbf_call2(s2.reshape(_E2 // 128, 128), mu1, ls1, out1)
    return out2.T


# speedup vs baseline: 1.0133x; 1.0133x over previous
import functools
import math

import jax
import jax.numpy as jnp
from jax import lax
from jax.experimental import pallas as pl
from jax.experimental.pallas import tpu as pltpu
from jax.experimental.pallas import tpu_sc as plsc

N_NODES = 10000
E = 640000
K = 64
NC = 2
NS = 16
NW = NC * NS

_BE = 25600
_NB1 = 6
_E1 = _NB1 * _BE
_E2 = E - _E1

_mesh = plsc.VectorSubcoreMesh(core_axis_name="c", subcore_axis_name="s")


def _make_sqdist_sc(e0, n_edges):
    epw = n_edges // NW
    tiles = (epw + 96 + 127) // 128

    @functools.partial(
        pl.kernel,
        mesh=_mesh,
        compiler_params=pltpu.CompilerParams(needs_layout_passes=False),
        out_type=jax.ShapeDtypeStruct((n_edges,), jnp.float32),
        scratch_types=[
            pltpu.VMEM((N_NODES,), jnp.float32),
            pltpu.VMEM((N_NODES,), jnp.float32),
            pltpu.VMEM((N_NODES,), jnp.float32),
            pltpu.VMEM((2 * 128 * tiles,), jnp.int32),
            pltpu.VMEM((epw,), jnp.float32),
            pltpu.SemaphoreType.DMA,
        ],
    )
    def _sqdist(posT_hbm, eit_hbm, out_hbm, px_v, py_v, pz_v, ei_v, out_v, sem):
        wid = lax.axis_index("s") * NC + lax.axis_index("c")
        base = wid * epw
        gbase = e0 + base
        t0 = gbase // 128
        ls = gbase - 128 * t0
        cps = [
            pltpu.async_copy(posT_hbm.at[pl.ds(0, N_NODES)], px_v, sem),
            pltpu.async_copy(posT_hbm.at[pl.ds(N_NODES, N_NODES)], py_v, sem),
            pltpu.async_copy(posT_hbm.at[pl.ds(2 * N_NODES, N_NODES)], pz_v, sem),
            pltpu.async_copy(eit_hbm.at[pl.ds(256 * t0, 256 * tiles)], ei_v, sem),
        ]
        for cp in cps:
            cp.wait()

        @plsc.parallel_loop(0, epw, step=16, unroll=8)
        def _body(l):
            le = ls + l
            b = le >> 7
            off = (b << 8) + (le - (b << 7))
            si = ei_v[pl.ds(off, 16)]
            di = ei_v[pl.ds(off + 128, 16)]
            dx = plsc.load_gather(px_v, [si]) - plsc.load_gather(px_v, [di])
            dy = plsc.load_gather(py_v, [si]) - plsc.load_gather(py_v, [di])
            dz = plsc.load_gather(pz_v, [si]) - plsc.load_gather(pz_v, [di])
            out_v[pl.ds(l, 16)] = dx * dx + dy * dy + dz * dz
        pltpu.sync_copy(out_v, out_hbm.at[pl.ds(base, epw)])

    return _sqdist


_sqdist_sc1 = _make_sqdist_sc(0, _E1)
_sqdist_sc2 = _make_sqdist_sc(_E1, _E2)

_R = _BE // 128


def _rbf_body(s_ref, mu_ref, ls_ref, out_ref):
    ls = ls_ref[...]
    sig = jnp.logaddexp(ls, 0.0)
    a = (-0.5 * math.log2(math.e)) / sig
    c = -1.0 / jnp.sqrt(2.0 * math.pi * sig)
    mub = jnp.broadcast_to(mu_ref[...], (K, 128))
    ab = jnp.broadcast_to(a, (K, 128))
    cb = jnp.broadcast_to(c, (K, 128))
    for r in range(_R):
        srow = s_ref[r:r + 1, :]
        d = srow * lax.rsqrt(srow + 1e-37)
        db = jnp.broadcast_to(d, (K, 128))
        diff = db - mub
        out_ref[:, r * 128:(r + 1) * 128] = cb * jnp.exp2(ab * (diff * diff))


def _rbf_phase1(s_ref, mu_ref, ls_ref, out_ref):
    _rbf_body(s_ref, mu_ref, ls_ref, out_ref)


def _rbf_phase2(s_ref, mu_ref, ls_ref, prev_ref, out_ref):
    del prev_ref
    _rbf_body(s_ref, mu_ref, ls_ref, out_ref)


_rbf_call1 = pl.pallas_call(
    _rbf_phase1,
    grid=(_NB1,),
    in_specs=[
        pl.BlockSpec((_R, 128), lambda i: (i, 0)),
        pl.BlockSpec((K, 1), lambda i: (0, 0)),
        pl.BlockSpec((K, 1), lambda i: (0, 0)),
    ],
    out_specs=pl.BlockSpec((K, _BE), lambda i: (0, i)),
    out_shape=jax.ShapeDtypeStruct((K, E), jnp.float32),
)

_rbf_call2 = pl.pallas_call(
    _rbf_phase2,
    grid=(_E2 // _BE,),
    in_specs=[
        pl.BlockSpec((_R, 128), lambda i: (i, 0)),
        pl.BlockSpec((K, 1), lambda i: (0, 0)),
        pl.BlockSpec((K, 1), lambda i: (0, 0)),
        pl.BlockSpec((8, 128), lambda i: (0, 0)),
    ],
    out_specs=pl.BlockSpec((K, _BE), lambda i: (0, i + _NB1)),
    out_shape=jax.ShapeDtypeStruct((K, E), jnp.float32),
    input_output_aliases={3: 0},
)


def kernel(edge_index, pos_matrix, mu, log_sigma):
    ei = edge_index.astype(jnp.int32)
    eit = ei.reshape(2, E // 128, 128).swapaxes(0, 1).reshape(2 * E)
    posT = pos_matrix.T.reshape(3 * N_NODES)
    mu1 = mu.reshape(K, 1)
    ls1 = log_sigma.reshape(K, 1)
    s1 = _sqdist_sc1(posT, eit)
    s2 = _sqdist_sc2(posT, eit)
    out1 = _rbf_call1(s1.reshape(_E1 // 128, 128), mu1, ls1)
    out2 = _rbf_call2(s2.reshape(_E2 // 128, 128), mu1, ls1, out1)
    return out2.T


# speedup vs baseline: 1.0851x; 1.0709x over previous
import functools
import math

import jax
import jax.numpy as jnp
from jax import lax
from jax.experimental import pallas as pl
from jax.experimental.pallas import tpu as pltpu
from jax.experimental.pallas import tpu_sc as plsc

N_NODES = 10000
E = 640000
K = 64
NC = 2
NS = 16
NW = NC * NS
EPW = E // NW

_mesh = plsc.VectorSubcoreMesh(core_axis_name="c", subcore_axis_name="s")

_TILES = 157


@functools.partial(
    pl.kernel,
    mesh=_mesh,
    compiler_params=pltpu.CompilerParams(needs_layout_passes=False),
    out_type=jax.ShapeDtypeStruct((E,), jnp.float32),
    scratch_types=[
        pltpu.VMEM((N_NODES,), jnp.float32),
        pltpu.VMEM((N_NODES,), jnp.float32),
        pltpu.VMEM((N_NODES,), jnp.float32),
        pltpu.VMEM((2 * 128 * _TILES,), jnp.int32),
        pltpu.VMEM((EPW,), jnp.float32),
        pltpu.SemaphoreType.DMA,
    ],
)
def _sqdist_sc(posT_hbm, eit_hbm, out_hbm, px_v, py_v, pz_v, ei_v, out_v, sem):
    wid = lax.axis_index("s") * NC + lax.axis_index("c")
    base = wid * EPW
    t0 = base // 128
    ls = base - 128 * t0
    cps = [
        pltpu.async_copy(posT_hbm.at[pl.ds(0, N_NODES)], px_v, sem),
        pltpu.async_copy(posT_hbm.at[pl.ds(N_NODES, N_NODES)], py_v, sem),
        pltpu.async_copy(posT_hbm.at[pl.ds(2 * N_NODES, N_NODES)], pz_v, sem),
        pltpu.async_copy(eit_hbm.at[pl.ds(256 * t0, 256 * _TILES)], ei_v, sem),
    ]
    for cp in cps:
        cp.wait()

    @plsc.parallel_loop(0, EPW, step=16, unroll=8)
    def _body(l):
        le = ls + l
        b = le >> 7
        off = (b << 8) + (le - (b << 7))
        si = ei_v[pl.ds(off, 16)]
        di = ei_v[pl.ds(off + 128, 16)]
        dx = plsc.load_gather(px_v, [si]) - plsc.load_gather(px_v, [di])
        dy = plsc.load_gather(py_v, [si]) - plsc.load_gather(py_v, [di])
        dz = plsc.load_gather(pz_v, [si]) - plsc.load_gather(pz_v, [di])
        out_v[pl.ds(l, 16)] = dx * dx + dy * dy + dz * dz
    pltpu.sync_copy(out_v, out_hbm.at[pl.ds(base, EPW)])


_R = 200
_BE = _R * 128


def _rbf_tc(s_ref, mu_ref, ls_ref, out_ref):
    ls = ls_ref[...]
    sig = jnp.logaddexp(ls, 0.0)
    a = (-0.5 * math.log2(math.e)) / sig
    c = -1.0 / jnp.sqrt(2.0 * math.pi * sig)
    mub = jnp.broadcast_to(mu_ref[...], (K, 128))
    ab = jnp.broadcast_to(a, (K, 128))
    cb = jnp.broadcast_to(c, (K, 128))
    for r in range(_R):
        srow = s_ref[r:r + 1, :]
        d = srow * lax.rsqrt(srow + 1e-37)
        db = jnp.broadcast_to(d, (K, 128))
        diff = db - mub
        out_ref[:, r * 128:(r + 1) * 128] = cb * jnp.exp2(ab * (diff * diff))


_rbf_call = pl.pallas_call(
    _rbf_tc,
    grid=(E // _BE,),
    in_specs=[
        pl.BlockSpec((_R, 128), lambda i: (i, 0)),
        pl.BlockSpec((K, 1), lambda i: (0, 0)),
        pl.BlockSpec((K, 1), lambda i: (0, 0)),
    ],
    out_specs=pl.BlockSpec((K, _BE), lambda i: (0, i)),
    out_shape=jax.ShapeDtypeStruct((K, E), jnp.float32),
)


def kernel(edge_index, pos_matrix, mu, log_sigma):
    ei = edge_index.astype(jnp.int32)
    eit = ei.reshape(2, E // 128, 128).swapaxes(0, 1).reshape(2 * E)
    posT = pos_matrix.T.reshape(3 * N_NODES)
    s = _sqdist_sc(posT, eit)
    s2d = s.reshape(E // 128, 128)
    outT = _rbf_call(s2d, mu.reshape(K, 1), log_sigma.reshape(K, 1))
    return outT.T
